# Initial kernel scaffold; baseline (speedup 1.0000x reference)
import jax
import jax.numpy as jnp
from jax.experimental import pallas as pl


def kernel(x, edge_index, sage_w_l, sage_b_l, sage_w_r, gat_w, gat_att_src, gat_att_dst, gat_bias):
    raise NotImplementedError("write your pallas kernel here")


# speedup vs baseline: 15.6942x; 15.6942x over previous
import functools

import jax
import jax.numpy as jnp
from jax import lax
from jax.experimental import pallas as pl
from jax.experimental.pallas import tpu as pltpu
from jax.experimental.pallas import tpu_sc as plsc

N = 10000
E = 320000
D = 128
NC, NS, L = 2, 16, 16
NW = NC * NS
E_PER_W = E // NW
CHUNK = 80
N_CHUNKS = E_PER_W // CHUNK
ROWS_PER_TILE = 624
ROWS_REM = N - NS * ROWS_PER_TILE


def _edge_sweep_body(weighted, *refs):
    if weighted:
        (rows_hbm, src_hbm, dst_hbm, zr_hbm, zc_hbm, as_hbm, ad_hbm,
         accp_hbm, denp_hbm,
         idx_s, idx_d, rows_v, wv, as_v, ad_v, acc_sh, den_sh, sem) = refs
    else:
        (rows_hbm, src_hbm, dst_hbm, zr_hbm, zc_hbm,
         accp_hbm, denp_hbm,
         idx_s, idx_d, rows_v, wv, acc_sh, den_sh, sem) = refs

    c = lax.axis_index("c")
    s = lax.axis_index("s")
    wid = s * NC + c

    pltpu.sync_copy(zr_hbm.at[pl.ds(s * ROWS_PER_TILE, ROWS_PER_TILE)],
                    acc_sh.at[pl.ds(s * ROWS_PER_TILE, ROWS_PER_TILE)])

    @pl.when(s == 0)
    def _():
        pltpu.sync_copy(zr_hbm.at[pl.ds(NS * ROWS_PER_TILE, ROWS_REM)],
                        acc_sh.at[pl.ds(NS * ROWS_PER_TILE, ROWS_REM)])
        pltpu.sync_copy(zc_hbm, den_sh)

    if weighted:
        pltpu.sync_copy(as_hbm, as_v)
        pltpu.sync_copy(ad_hbm, ad_v)
    else:
        for g in range(CHUNK // L):
            wv[pl.ds(g * L, L)] = jnp.ones((L,), jnp.float32)

    plsc.subcore_barrier()

    base_w = wid * E_PER_W

    def chunk_body(ci, carry):
        base = base_w + ci * CHUNK
        pltpu.sync_copy(src_hbm.at[pl.ds(base, CHUNK)], idx_s)
        pltpu.sync_copy(dst_hbm.at[pl.ds(base, CHUNK)], idx_d)
        pltpu.async_copy(rows_hbm.at[idx_s], rows_v, sem).wait()
        if weighted:
            for g in range(CHUNK // L):
                sv = idx_s[pl.ds(g * L, L)]
                dv = idx_d[pl.ds(g * L, L)]
                e = plsc.load_gather(as_v, [sv]) + plsc.load_gather(ad_v, [dv])
                e = jnp.where(e > 0, e, 0.2 * e)
                wv[pl.ds(g * L, L)] = jnp.exp(e)

            def group_body(g, carry2):
                wgroup = wv[pl.ds(g * L, L)]
                for j16 in range(L):
                    w = wgroup[j16]
                    j = g * L + j16
                    for r in range(D // L):
                        rows_v[j, pl.ds(r * L, L)] = rows_v[j, pl.ds(r * L, L)] * w
                return carry2

            lax.fori_loop(0, CHUNK // L, group_body, 0)
        pltpu.sync_copy(rows_v, acc_sh.at[idx_d], add=True)
        pltpu.sync_copy(wv, den_sh.at[idx_d], add=True)
        return carry

    lax.fori_loop(0, N_CHUNKS, chunk_body, 0)

    plsc.subcore_barrier()

    pltpu.sync_copy(acc_sh.at[pl.ds(s * ROWS_PER_TILE, ROWS_PER_TILE)],
                    accp_hbm.at[c, pl.ds(s * ROWS_PER_TILE, ROWS_PER_TILE)])

    @pl.when(s == 0)
    def _():
        pltpu.sync_copy(acc_sh.at[pl.ds(NS * ROWS_PER_TILE, ROWS_REM)],
                        accp_hbm.at[c, pl.ds(NS * ROWS_PER_TILE, ROWS_REM)])
        pltpu.sync_copy(den_sh, denp_hbm.at[c])


def _edge_sweep(weighted, rows, src, dst, a_s=None, a_d=None):
    mesh = plsc.VectorSubcoreMesh(core_axis_name="c", subcore_axis_name="s",
                                  num_cores=NC, num_subcores=NS)
    scratch = [
        pltpu.VMEM((CHUNK,), jnp.int32),
        pltpu.VMEM((CHUNK,), jnp.int32),
        pltpu.VMEM((CHUNK, D), jnp.float32),
        pltpu.VMEM((CHUNK,), jnp.float32),
    ]
    if weighted:
        scratch += [
            pltpu.VMEM((N,), jnp.float32),
            pltpu.VMEM((N,), jnp.float32),
        ]
    scratch += [
        pltpu.VMEM_SHARED((N, D), jnp.float32),
        pltpu.VMEM_SHARED((N,), jnp.float32),
        pltpu.SemaphoreType.DMA,
    ]
    fn = pl.kernel(
        functools.partial(_edge_sweep_body, weighted),
        out_type=(
            jax.ShapeDtypeStruct((NC, N, D), jnp.float32),
            jax.ShapeDtypeStruct((NC, N), jnp.float32),
        ),
        mesh=mesh,
        scratch_types=scratch,
        compiler_params=pltpu.CompilerParams(needs_layout_passes=False),
    )
    zr = jnp.zeros((N, D), jnp.float32)
    zc = jnp.zeros((N,), jnp.float32)
    if weighted:
        return fn(rows, src, dst, zr, zc, a_s, a_d)
    return fn(rows, src, dst, zr, zc)


def _dense_mid_body(accp, cntp, x, wl, wr, gw, bl, ats, atd, hg_o, as_o, ad_o):
    agg = accp[0] + accp[1]
    cnt = cntp[0] + cntp[1]
    mean = agg / jnp.maximum(cnt, 1.0)
    dn = (((1,), (1,)), ((), ()))
    sage = (lax.dot_general(mean, wl[...], dn, preferred_element_type=jnp.float32)
            + bl[...]
            + lax.dot_general(x[...], wr[...], dn, preferred_element_type=jnp.float32))
    h = jnp.maximum(sage, 0.0)
    hg = lax.dot_general(h, gw[...], dn, preferred_element_type=jnp.float32)
    hg_o[...] = hg
    as_o[...] = jnp.sum(hg * ats[...], axis=1, keepdims=True)
    ad_o[...] = jnp.sum(hg * atd[...], axis=1, keepdims=True)


def _dense_final_body(nump, denp, hg, a_s, a_d, bias, out_o):
    num = nump[0] + nump[1]
    den = denp[0] + denp[1]
    es = a_s[...] + a_d[...]
    es = jnp.where(es > 0, es, 0.2 * es)
    ws = jnp.exp(es)
    num = num + ws * hg[...]
    den = den + ws
    o = num / (den + 1e-16) + bias[...]
    m = jnp.max(o, axis=1, keepdims=True)
    lse = jnp.log(jnp.sum(jnp.exp(o - m), axis=1, keepdims=True)) + m
    out_o[...] = o - lse


_BR = 400
_GRID = N // _BR


def _dense_mid(accp, cntp, x, wl, wr, gw, bl, ats, atd):
    return pl.pallas_call(
        _dense_mid_body,
        grid=(_GRID,),
        in_specs=[
            pl.BlockSpec((NC, _BR, D), lambda i: (0, i, 0)),
            pl.BlockSpec((NC, _BR, 1), lambda i: (0, i, 0)),
            pl.BlockSpec((_BR, D), lambda i: (i, 0)),
            pl.BlockSpec((D, D), lambda i: (0, 0)),
            pl.BlockSpec((D, D), lambda i: (0, 0)),
            pl.BlockSpec((D, D), lambda i: (0, 0)),
            pl.BlockSpec((1, D), lambda i: (0, 0)),
            pl.BlockSpec((1, D), lambda i: (0, 0)),
            pl.BlockSpec((1, D), lambda i: (0, 0)),
        ],
        out_specs=[
            pl.BlockSpec((_BR, D), lambda i: (i, 0)),
            pl.BlockSpec((_BR, 1), lambda i: (i, 0)),
            pl.BlockSpec((_BR, 1), lambda i: (i, 0)),
        ],
        out_shape=[
            jax.ShapeDtypeStruct((N, D), jnp.float32),
            jax.ShapeDtypeStruct((N, 1), jnp.float32),
            jax.ShapeDtypeStruct((N, 1), jnp.float32),
        ],
    )(accp, cntp, x, wl, wr, gw, bl, ats, atd)


def _dense_final(nump, denp, hg, a_s, a_d, bias):
    return pl.pallas_call(
        _dense_final_body,
        grid=(_GRID,),
        in_specs=[
            pl.BlockSpec((NC, _BR, D), lambda i: (0, i, 0)),
            pl.BlockSpec((NC, _BR, 1), lambda i: (0, i, 0)),
            pl.BlockSpec((_BR, D), lambda i: (i, 0)),
            pl.BlockSpec((_BR, 1), lambda i: (i, 0)),
            pl.BlockSpec((_BR, 1), lambda i: (i, 0)),
            pl.BlockSpec((1, D), lambda i: (0, 0)),
        ],
        out_specs=pl.BlockSpec((_BR, D), lambda i: (i, 0)),
        out_shape=jax.ShapeDtypeStruct((N, D), jnp.float32),
    )(nump, denp, hg, a_s, a_d, bias)


def kernel(x, edge_index, sage_w_l, sage_b_l, sage_w_r, gat_w, gat_att_src,
           gat_att_dst, gat_bias):
    src = edge_index[0].astype(jnp.int32)
    dst = edge_index[1].astype(jnp.int32)

    accp, cntp = _edge_sweep(False, x, src, dst)

    hg, a_s, a_d = _dense_mid(
        accp, cntp.reshape(NC, N, 1), x,
        sage_w_l, sage_w_r, gat_w,
        sage_b_l.reshape(1, D),
        gat_att_src.reshape(1, D), gat_att_dst.reshape(1, D))

    nump, denp = _edge_sweep(True, hg, src, dst,
                             a_s.reshape(N), a_d.reshape(N))

    return _dense_final(nump, denp.reshape(NC, N, 1), hg, a_s, a_d,
                        gat_bias.reshape(1, D))
